# Initial kernel scaffold; baseline (speedup 1.0000x reference)
import jax
import jax.numpy as jnp
from jax.experimental import pallas as pl


def kernel(indices, table, W1, b1, W2, b2):
    raise NotImplementedError("write your pallas kernel here")


# speedup vs baseline: 7.4289x; 7.4289x over previous
import functools

import jax
import jax.numpy as jnp
from jax import lax
from jax.experimental import pallas as pl
from jax.experimental.pallas import tpu as pltpu
from jax.experimental.pallas import tpu_sc as plsc

B = 16384
L = 200
D = 32
NW = 32
BPW = B // NW


def _pool_body(idx_hbm, table_hbm, pooled_hbm, idx_v, rows_v, pooled_v, sem):
    wid = lax.axis_index("s") * 2 + lax.axis_index("c")
    base = wid * BPW

    def row_loop(r, _):
        pltpu.sync_copy(idx_hbm.at[pl.ds((base + r) * L, L)], idx_v)
        pltpu.async_copy(table_hbm.at[idx_v], rows_v, sem).wait()

        def seq_body(j, carry):
            lo, hi = carry
            lo = lo + rows_v[j, pl.ds(0, 16)]
            hi = hi + rows_v[j, pl.ds(16, 16)]
            return lo, hi

        zero = jnp.zeros((16,), jnp.float32)
        lo, hi = lax.fori_loop(0, L, seq_body, (zero, zero))
        scale = jnp.float32(1.0 / L)
        pooled_v[r, pl.ds(0, 16)] = lo * scale
        pooled_v[r, pl.ds(16, 16)] = hi * scale
        return 0

    lax.fori_loop(0, BPW, row_loop, 0)
    pltpu.sync_copy(pooled_v, pooled_hbm.at[pl.ds(base, BPW)])


_pool = functools.partial(
    pl.kernel,
    mesh=plsc.VectorSubcoreMesh(core_axis_name="c", subcore_axis_name="s"),
    compiler_params=pltpu.CompilerParams(use_tc_tiling_on_sc=False),
    out_type=jax.ShapeDtypeStruct((B, D), jnp.float32),
    scratch_types=[
        pltpu.VMEM((L,), jnp.int32),
        pltpu.VMEM((L, D), jnp.float32),
        pltpu.VMEM((BPW, D), jnp.float32),
        pltpu.SemaphoreType.DMA,
    ],
)(_pool_body)


def _mlp_body(x_ref, w1_ref, b1_ref, w2_ref, b2_ref, out_ref):
    x = x_ref[...]
    h = jnp.dot(x, w1_ref[...], preferred_element_type=jnp.float32) + b1_ref[...]
    h = jnp.maximum(h, 0.0)
    out_ref[...] = jnp.dot(h, w2_ref[...], preferred_element_type=jnp.float32) + b2_ref[...]


def _mlp(pooled, W1, b1, W2, b2):
    return pl.pallas_call(
        _mlp_body,
        out_shape=jax.ShapeDtypeStruct((B, 1), jnp.float32),
    )(pooled, W1, b1.reshape(1, -1), W2, b2.reshape(1, -1))


def kernel(indices, table, W1, b1, W2, b2):
    idx_flat = indices.astype(jnp.int32).reshape(-1)
    pooled = _pool(idx_flat, table)
    return _mlp(pooled, W1, b1, W2, b2)


# speedup vs baseline: 16.2016x; 2.1809x over previous
import functools

import jax
import jax.numpy as jnp
from jax import lax
from jax.experimental import pallas as pl
from jax.experimental.pallas import tpu as pltpu
from jax.experimental.pallas import tpu_sc as plsc

B = 16384
L = 200
D = 32
NW = 32
BPW = B // NW
CH = 40
NCH = L // CH


def _pool_body(idxt_hbm, table_hbm, pooled_hbm,
               idx_a, idx_b, acc0, acc1, sem_i, sem0, sem1):
    wid = lax.axis_index("s") * 2 + lax.axis_index("c")
    base = wid * BPW
    idx_bufs = (idx_a, idx_b)
    sems = (sem_i, sem_i)

    def idx_fetch(c, buf):
        return pltpu.async_copy(
            idxt_hbm.at[pl.ds(c * CH, CH), pl.ds(base, BPW)], buf, sem_i)

    idx_fetch(0, idx_a).wait()
    fetch1 = idx_fetch(1, idx_b)

    pltpu.async_copy(table_hbm.at[idx_a.at[0]], acc0, sem0)
    pltpu.async_copy(table_hbm.at[idx_a.at[1]], acc1, sem1)

    def make_pair_body(idx_buf):
        def pair_body(k, _):
            row0 = idx_buf.at[2 * k]
            row1 = idx_buf.at[2 * k + 1]
            pltpu.make_async_copy(table_hbm.at[row0], acc0, sem0).wait()
            pltpu.async_copy(table_hbm.at[row0], acc0, sem0, add=True)
            pltpu.make_async_copy(table_hbm.at[row1], acc1, sem1).wait()
            pltpu.async_copy(table_hbm.at[row1], acc1, sem1, add=True)
            return 0
        return pair_body

    lax.fori_loop(1, CH // 2, make_pair_body(idx_a), 0)

    pending = fetch1
    for c in range(1, NCH):
        buf = idx_bufs[c % 2]
        pending.wait()
        if c + 1 < NCH:
            pending = idx_fetch(c + 1, idx_bufs[(c + 1) % 2])
        lax.fori_loop(0, CH // 2, make_pair_body(buf), 0)

    pltpu.make_async_copy(table_hbm.at[idx_a.at[0]], acc0, sem0).wait()
    pltpu.make_async_copy(table_hbm.at[idx_a.at[1]], acc1, sem1).wait()

    def comb_body(r, _):
        acc0[r, pl.ds(0, 16)] = acc0[r, pl.ds(0, 16)] + acc1[r, pl.ds(0, 16)]
        acc0[r, pl.ds(16, 16)] = acc0[r, pl.ds(16, 16)] + acc1[r, pl.ds(16, 16)]
        return 0

    lax.fori_loop(0, BPW, comb_body, 0)
    pltpu.sync_copy(acc0, pooled_hbm.at[pl.ds(base, BPW)])


_pool = functools.partial(
    pl.kernel,
    mesh=plsc.VectorSubcoreMesh(core_axis_name="c", subcore_axis_name="s"),
    compiler_params=pltpu.CompilerParams(use_tc_tiling_on_sc=False),
    out_type=jax.ShapeDtypeStruct((B, D), jnp.float32),
    scratch_types=[
        pltpu.VMEM((CH, BPW), jnp.int32),
        pltpu.VMEM((CH, BPW), jnp.int32),
        pltpu.VMEM((BPW, D), jnp.float32),
        pltpu.VMEM((BPW, D), jnp.float32),
        pltpu.SemaphoreType.DMA,
        pltpu.SemaphoreType.DMA,
        pltpu.SemaphoreType.DMA,
    ],
)(_pool_body)


def _mlp_body(x_ref, w1_ref, b1_ref, w2_ref, b2_ref, out_ref):
    x = x_ref[...] * jnp.float32(1.0 / L)
    h = jnp.dot(x, w1_ref[...], preferred_element_type=jnp.float32) + b1_ref[...]
    h = jnp.maximum(h, 0.0)
    out_ref[...] = jnp.dot(h, w2_ref[...], preferred_element_type=jnp.float32) + b2_ref[...]


def _mlp(pooled, W1, b1, W2, b2):
    return pl.pallas_call(
        _mlp_body,
        out_shape=jax.ShapeDtypeStruct((B, 1), jnp.float32),
    )(pooled, W1, b1.reshape(1, -1), W2, b2.reshape(1, -1))


def kernel(indices, table, W1, b1, W2, b2):
    idx_t = indices.astype(jnp.int32).T
    pooled = _pool(idx_t, table)
    return _mlp(pooled, W1, b1, W2, b2)
